# Initial kernel scaffold; baseline (speedup 1.0000x reference)
import functools

import jax
import jax.numpy as jnp
from jax import lax
from jax.experimental import pallas as pl
from jax.experimental.pallas import tpu as pltpu
from jax.experimental.pallas import tpu_sc as plsc

N = 10000
E = 320000
F = 128
H = 128
C = 40

NC = 2
NS = 16
L = 16
NW = NC * NS
EPW = E // NW
CH = 80
NCHUNK = EPW // CH
RPT = N // NS
ZR = 125
CNTW = 16


def _sc_aggregate(x, src3, dst3):
    mesh = plsc.VectorSubcoreMesh(core_axis_name="c", subcore_axis_name="s")

    @functools.partial(
        pl.kernel,
        out_type=(
            jax.ShapeDtypeStruct((NC, N, F), jnp.float32),
            jax.ShapeDtypeStruct((NC, N, CNTW), jnp.float32),
        ),
        mesh=mesh,
        scratch_types=[
            pltpu.VMEM((NCHUNK, CH), jnp.int32),
            pltpu.VMEM((NCHUNK, CH), jnp.int32),
            pltpu.VMEM((CH, F), jnp.float32),
            pltpu.VMEM((CH, CNTW), jnp.float32),
            pltpu.VMEM((ZR, F), jnp.float32),
            pltpu.VMEM((RPT, CNTW), jnp.float32),
            pltpu.VMEM_SHARED((N, F), jnp.float32),
            pltpu.VMEM_SHARED((N, CNTW), jnp.float32),
        ],
    )
    def agg(x_hbm, src_hbm, dst_hbm, acc_out, cnt_out,
            sidx, didx, rows, ones_v, zacc, zcnt, acc_sh, cnt_sh):
        cid = lax.axis_index("c")
        sid = lax.axis_index("s")
        wid = sid * NC + cid

        zv = jnp.zeros((L,), jnp.float32)
        ov = jnp.ones((L,), jnp.float32)

        @pl.loop(0, ZR)
        def _(i):
            for j in range(F // L):
                zacc[i, pl.ds(j * L, L)] = zv

        @pl.loop(0, RPT)
        def _(i):
            zcnt[i, pl.ds(0, L)] = zv

        @pl.loop(0, CH)
        def _(i):
            ones_v[i, pl.ds(0, L)] = ov

        r0 = sid * RPT
        for t in range(RPT // ZR):
            pltpu.sync_copy(zacc, acc_sh.at[pl.ds(r0 + t * ZR, ZR)])
        pltpu.sync_copy(zcnt, cnt_sh.at[pl.ds(r0, RPT)])

        pltpu.sync_copy(src_hbm.at[wid], sidx)
        pltpu.sync_copy(dst_hbm.at[wid], didx)

        plsc.subcore_barrier()

        @pl.loop(0, NCHUNK)
        def _(k):
            pltpu.sync_copy(x_hbm.at[sidx.at[k]], rows)
            pltpu.sync_copy(rows, acc_sh.at[didx.at[k]], add=True)
            pltpu.sync_copy(ones_v, cnt_sh.at[didx.at[k]], add=True)

        plsc.subcore_barrier()

        pltpu.sync_copy(acc_sh.at[pl.ds(r0, RPT)],
                        acc_out.at[cid, pl.ds(r0, RPT)])
        pltpu.sync_copy(cnt_sh.at[pl.ds(r0, RPT)],
                        cnt_out.at[cid, pl.ds(r0, RPT)])

    return agg(x, src3, dst3)


def _tc_dense(x, acc, cnt, W_self, W_neigh, b2, W_fc, bf2):
    R = 1250

    def body(x_ref, p_ref, c_ref, ws_ref, wn_ref, b_ref, wf_ref, bf_ref,
             o_ref):
        dot = functools.partial(jnp.dot,
                                preferred_element_type=jnp.float32,
                                precision=lax.Precision.HIGHEST)
        s = p_ref[0] + p_ref[1]
        deg = c_ref[0, :, 0] + c_ref[1, :, 0]
        neigh = s / jnp.maximum(deg, 1.0)[:, None]
        h = dot(x_ref[...], ws_ref[...]) + dot(neigh, wn_ref[...]) + b_ref[...]
        h = jnp.maximum(h, 0.0)
        o_ref[...] = dot(h, wf_ref[...]) + bf_ref[...]

    return pl.pallas_call(
        body,
        grid=(N // R,),
        in_specs=[
            pl.BlockSpec((R, F), lambda i: (i, 0)),
            pl.BlockSpec((NC, R, F), lambda i: (0, i, 0)),
            pl.BlockSpec((NC, R, CNTW), lambda i: (0, i, 0)),
            pl.BlockSpec((F, H), lambda i: (0, 0)),
            pl.BlockSpec((F, H), lambda i: (0, 0)),
            pl.BlockSpec((1, H), lambda i: (0, 0)),
            pl.BlockSpec((H, C), lambda i: (0, 0)),
            pl.BlockSpec((1, C), lambda i: (0, 0)),
        ],
        out_specs=pl.BlockSpec((R, C), lambda i: (i, 0)),
        out_shape=jax.ShapeDtypeStruct((N, C), jnp.float32),
    )(x, acc, cnt, W_self, W_neigh, b2, W_fc, bf2)


def kernel(x, edge_index, W_self, W_neigh, b, W_fc, b_fc):
    src3 = edge_index[0].astype(jnp.int32).reshape(NW, NCHUNK, CH)
    dst3 = edge_index[1].astype(jnp.int32).reshape(NW, NCHUNK, CH)
    acc, cnt = _sc_aggregate(x, src3, dst3)
    return _tc_dense(x, acc, cnt, W_self, W_neigh,
                     b.reshape(1, H), W_fc, b_fc.reshape(1, C))


# speedup vs baseline: 8.2340x; 8.2340x over previous
import functools

import jax
import jax.numpy as jnp
from jax import lax
from jax.experimental import pallas as pl
from jax.experimental.pallas import tpu as pltpu
from jax.experimental.pallas import tpu_sc as plsc

N = 10000
E = 320000
F = 128
H = 128
C = 40

NC = 2
NS = 16
L = 16
NW = NC * NS
EPW = E // NW
CH = 125
NCHUNK = EPW // CH
RPT = N // NS
CNTW = 16


def _sc_aggregate(x, src3, dst3):
    mesh = plsc.VectorSubcoreMesh(core_axis_name="c", subcore_axis_name="s")

    @functools.partial(
        pl.kernel,
        out_type=(
            jax.ShapeDtypeStruct((NC, N, F), jnp.float32),
            jax.ShapeDtypeStruct((NC, N, CNTW), jnp.float32),
        ),
        mesh=mesh,
        compiler_params=pltpu.CompilerParams(use_tc_tiling_on_sc=False),
        scratch_types=[
            pltpu.VMEM((NCHUNK, CH), jnp.int32),
            pltpu.VMEM((NCHUNK, CH), jnp.int32),
            pltpu.VMEM((CH, F), jnp.float32),
            pltpu.VMEM((CH, CNTW), jnp.float32),
            pltpu.VMEM_SHARED((N, F), jnp.float32),
            pltpu.VMEM_SHARED((N, CNTW), jnp.float32),
        ],
    )
    def agg(x_hbm, src_hbm, dst_hbm, acc_out, cnt_out,
            sidx, didx, rows, ones_v, acc_sh, cnt_sh):
        cid = lax.axis_index("c")
        sid = lax.axis_index("s")
        wid = sid * NC + cid

        zv = jnp.zeros((L,), jnp.float32)
        ov = jnp.ones((L,), jnp.float32)

        @pl.loop(0, CH)
        def _(i):
            for j in range(F // L):
                rows[i, pl.ds(j * L, L)] = zv
            ones_v[i, pl.ds(0, L)] = zv

        r0 = sid * RPT
        for t in range(RPT // CH):
            pltpu.sync_copy(rows, acc_sh.at[pl.ds(r0 + t * CH, CH)])
            pltpu.sync_copy(ones_v, cnt_sh.at[pl.ds(r0 + t * CH, CH)])

        @pl.loop(0, CH)
        def _(i):
            ones_v[i, pl.ds(0, L)] = ov

        pltpu.sync_copy(src_hbm.at[wid], sidx)
        pltpu.sync_copy(dst_hbm.at[wid], didx)

        plsc.subcore_barrier()

        @pl.loop(0, NCHUNK)
        def _(k):
            pltpu.sync_copy(x_hbm.at[sidx.at[k]], rows)
            pltpu.sync_copy(rows, acc_sh.at[didx.at[k]], add=True)
            pltpu.sync_copy(ones_v, cnt_sh.at[didx.at[k]], add=True)

        plsc.subcore_barrier()

        pltpu.sync_copy(acc_sh.at[pl.ds(r0, RPT)],
                        acc_out.at[cid, pl.ds(r0, RPT)])
        pltpu.sync_copy(cnt_sh.at[pl.ds(r0, RPT)],
                        cnt_out.at[cid, pl.ds(r0, RPT)])

    return agg(x, src3, dst3)


def _tc_dense(x, acc, cnt, W_self, W_neigh, b2, W_fc, bf2):
    R = 1000

    def body(x_ref, p_ref, c_ref, ws_ref, wn_ref, b_ref, wf_ref, bf_ref,
             o_ref):
        dot = functools.partial(jnp.dot,
                                preferred_element_type=jnp.float32,
                                precision=lax.Precision.HIGHEST)
        s = p_ref[0] + p_ref[1]
        deg = c_ref[0, :, 0] + c_ref[1, :, 0]
        neigh = s / jnp.maximum(deg, 1.0)[:, None]
        h = dot(x_ref[...], ws_ref[...]) + dot(neigh, wn_ref[...]) + b_ref[...]
        h = jnp.maximum(h, 0.0)
        o_ref[...] = dot(h, wf_ref[...]) + bf_ref[...]

    return pl.pallas_call(
        body,
        grid=(N // R,),
        in_specs=[
            pl.BlockSpec((R, F), lambda i: (i, 0)),
            pl.BlockSpec((NC, R, F), lambda i: (0, i, 0)),
            pl.BlockSpec((NC, R, CNTW), lambda i: (0, i, 0)),
            pl.BlockSpec((F, H), lambda i: (0, 0)),
            pl.BlockSpec((F, H), lambda i: (0, 0)),
            pl.BlockSpec((1, H), lambda i: (0, 0)),
            pl.BlockSpec((H, C), lambda i: (0, 0)),
            pl.BlockSpec((1, C), lambda i: (0, 0)),
        ],
        out_specs=pl.BlockSpec((R, C), lambda i: (i, 0)),
        out_shape=jax.ShapeDtypeStruct((N, C), jnp.float32),
    )(x, acc, cnt, W_self, W_neigh, b2, W_fc, bf2)


def kernel(x, edge_index, W_self, W_neigh, b, W_fc, b_fc):
    src3 = edge_index[0].astype(jnp.int32).reshape(NW, NCHUNK, CH)
    dst3 = edge_index[1].astype(jnp.int32).reshape(NW, NCHUNK, CH)
    acc, cnt = _sc_aggregate(x, src3, dst3)
    return _tc_dense(x, acc, cnt, W_self, W_neigh,
                     b.reshape(1, H), W_fc, b_fc.reshape(1, C))
